# Initial kernel scaffold; baseline (speedup 1.0000x reference)
import jax
import jax.numpy as jnp
from jax.experimental import pallas as pl


def kernel(x, edge_index, all_negative_edges, W1, W2, Wd1, bd1, Wd2, bd2):
    raise NotImplementedError("write your pallas kernel here")


# speedup vs baseline: 1.9297x; 1.9297x over previous
import functools

import jax
import jax.numpy as jnp
from jax import lax
from jax.experimental import pallas as pl
from jax.experimental.pallas import tpu as pltpu
from jax.experimental.pallas import tpu_sc as plsc

N_NODES = 10000
NP = 10240
D = 128
PAD_NODE = 10200
E_REM = 96000
E_REM_PAD = 98304
N_MASK = 224000
HALF_PAD = 229376
DEC_TOTAL = 458752
NWORK = 32
C_PER_TILE_REM = 24
C_PER_TILE_NEG = 56
C_PER_TILE_DEC = 112
ROWS_PER_TILE = NP // 16

_MESH = plsc.VectorSubcoreMesh(
    core_axis_name="c", subcore_axis_name="s", num_cores=2, num_subcores=16)


def _wid():
    return lax.axis_index("s") * 2 + lax.axis_index("c")


@functools.partial(
    pl.kernel,
    out_type=(
        jax.ShapeDtypeStruct((2 * NP,), jnp.float32),
        jax.ShapeDtypeStruct((HALF_PAD,), jnp.int32),
        jax.ShapeDtypeStruct((HALF_PAD,), jnp.int32),
    ),
    mesh=_MESH,
    scratch_types=(
        pltpu.VMEM((128,), jnp.int32),
        pltpu.VMEM((128,), jnp.int32),
        pltpu.VMEM((128,), jnp.int32),
        pltpu.VMEM((128,), jnp.float32),
        pltpu.VMEM((ROWS_PER_TILE,), jnp.float32),
        pltpu.VMEM_SHARED((NP,), jnp.float32),
        pltpu.SemaphoreType.DMA,
    ),
)
def _sc_deg_neg(rem_dst, neg_idx, nsrc_t, ndst_t, zflat,
                deg_out, nsrc_o, ndst_o,
                idx_v, val_a, val_b, ones_v, zb, deg_sh, sem):
    c = lax.axis_index("c")
    s = lax.axis_index("s")
    wid = _wid()
    for i in range(8):
        ones_v[pl.ds(i * 16, 16)] = jnp.ones((16,), jnp.float32)
    pltpu.sync_copy(zflat, zb)
    pltpu.sync_copy(zb, deg_sh.at[pl.ds(s * ROWS_PER_TILE, ROWS_PER_TILE)])
    plsc.subcore_barrier()
    for k in range(C_PER_TILE_REM):
        base = wid * 3072 + k * 128
        pltpu.sync_copy(rem_dst.at[pl.ds(base, 128)], idx_v)
        pltpu.sync_copy(ones_v, deg_sh.at[idx_v], add=True)

    def nbody(k, carry):
        base = wid * 7168 + k * 128
        pltpu.sync_copy(neg_idx.at[pl.ds(base, 128)], idx_v)
        a = pltpu.async_copy(nsrc_t.at[idx_v], val_a, sem)
        b = pltpu.async_copy(ndst_t.at[idx_v], val_b, sem)
        a.wait()
        b.wait()
        pltpu.sync_copy(val_a, nsrc_o.at[pl.ds(base, 128)])
        pltpu.sync_copy(val_b, ndst_o.at[pl.ds(base, 128)])
        return carry

    lax.fori_loop(0, C_PER_TILE_NEG, nbody, 0)
    plsc.subcore_barrier()
    pltpu.sync_copy(deg_sh.at[pl.ds(s * ROWS_PER_TILE, ROWS_PER_TILE)], zb)
    pltpu.sync_copy(
        zb, deg_out.at[pl.ds(c * NP + s * ROWS_PER_TILE, ROWS_PER_TILE)])


@functools.partial(
    pl.kernel,
    out_type=jax.ShapeDtypeStruct((2 * NP, D), jnp.float32),
    mesh=_MESH,
    scratch_types=(
        pltpu.VMEM((128,), jnp.int32),
        pltpu.VMEM((128,), jnp.int32),
        pltpu.VMEM((128, D), jnp.float32),
        pltpu.VMEM((128, D), jnp.float32),
        pltpu.VMEM_SHARED((NP, D), jnp.float32),
        pltpu.SemaphoreType.DMA,
    ),
)
def _sc_scatter_rows(hn, src_p, dst_p, z2d,
                     tmp_out,
                     idx_s, idx_d, rows_v, zb2, acc_sh, sem):
    c = lax.axis_index("c")
    s = lax.axis_index("s")
    wid = _wid()
    pltpu.sync_copy(z2d, zb2)
    for j in range(5):
        pltpu.sync_copy(
            zb2, acc_sh.at[pl.ds(s * ROWS_PER_TILE + j * 128, 128), :])
    plsc.subcore_barrier()
    for k in range(C_PER_TILE_REM):
        base = wid * 3072 + k * 128
        pltpu.sync_copy(src_p.at[pl.ds(base, 128)], idx_s)
        pltpu.sync_copy(dst_p.at[pl.ds(base, 128)], idx_d)
        pltpu.async_copy(hn.at[idx_s], rows_v, sem).wait()
        pltpu.sync_copy(rows_v, acc_sh.at[idx_d], add=True)
    plsc.subcore_barrier()
    for j in range(5):
        r0 = s * ROWS_PER_TILE + j * 128
        pltpu.sync_copy(acc_sh.at[pl.ds(r0, 128), :], zb2)
        pltpu.sync_copy(zb2, tmp_out.at[pl.ds(c * NP + r0, 128), :])


@functools.partial(
    pl.kernel,
    out_type=(
        jax.ShapeDtypeStruct((DEC_TOTAL, D), jnp.float32),
        jax.ShapeDtypeStruct((DEC_TOTAL, D), jnp.float32),
    ),
    mesh=_MESH,
    scratch_types=(
        pltpu.VMEM((128,), jnp.int32),
        pltpu.VMEM((128,), jnp.int32),
        pltpu.VMEM((128, D), jnp.float32),
        pltpu.VMEM((128, D), jnp.float32),
        pltpu.SemaphoreType.DMA,
        pltpu.SemaphoreType.DMA,
    ),
)
def _sc_dec_gather(z, dsrc, ddst,
                   fs, fd,
                   idx_s, idx_d, rows_s, rows_d, sem_g, sem_w):
    wid = _wid()

    def fbody(k, carry):
        base = wid * 14336 + k * 128
        pltpu.sync_copy(dsrc.at[pl.ds(base, 128)], idx_s)
        pltpu.sync_copy(ddst.at[pl.ds(base, 128)], idx_d)
        a = pltpu.async_copy(z.at[idx_s], rows_s, sem_g)
        b = pltpu.async_copy(z.at[idx_d], rows_d, sem_g)
        a.wait()
        b.wait()
        u = pltpu.async_copy(rows_s, fs.at[pl.ds(base, 128), :], sem_w)
        v = pltpu.async_copy(rows_d, fd.at[pl.ds(base, 128), :], sem_w)
        u.wait()
        v.wait()
        return carry

    lax.fori_loop(0, C_PER_TILE_DEC, fbody, 0)


def _tc_norm_body(d0, d1, x_ref, dinv_ref, hn_ref):
    deg = d0[...] + d1[...] + 1.0
    dinv = lax.rsqrt(deg)
    dinv_ref[...] = dinv
    hn_ref[...] = x_ref[...] * dinv


def _tc_layer_body(relu, t0, t1, hnp, dinv, w_ref, out_ref):
    agg = (t0[...] + t1[...] + hnp[...]) * dinv[...]
    r = jnp.dot(agg, w_ref[...], preferred_element_type=jnp.float32)
    if relu:
        r = jnp.maximum(r, 0.0)
        out_ref[...] = r * dinv[...]
    else:
        out_ref[...] = r


def _tc_dec_body(fs, fd, wd1, bd1, wd2, bd2, out_ref, acc):
    i = pl.program_id(0)

    @pl.when(i == 0)
    def _():
        acc[...] = jnp.zeros_like(acc)

    feat = fs[...] * fd[...]
    h = jnp.dot(feat, wd1[...], preferred_element_type=jnp.float32) + bd1[...]
    h = jnp.maximum(h, 0.0)
    logit = jnp.dot(h, wd2[...], preferred_element_type=jnp.float32) + bd2[...]
    sig = jax.nn.sigmoid(logit)
    rows = i * 1024 + lax.broadcasted_iota(jnp.int32, (1024, 1), 0)
    p = jnp.log1p(jnp.exp(-sig))
    n = jnp.log1p(jnp.exp(sig))
    sp = jnp.sum(jnp.where(rows < N_MASK, p, 0.0))
    sn = jnp.sum(jnp.where(
        (rows >= HALF_PAD) & (rows < HALF_PAD + N_MASK), n, 0.0))
    acc[...] += jnp.concatenate(
        [sp.reshape(1, 1), sn.reshape(1, 1)], axis=1)

    @pl.when(i == pl.num_programs(0) - 1)
    def _():
        out_ref[...] = jnp.sum(acc[...]).reshape(1, 1) / float(N_MASK)


def kernel(x, edge_index, all_negative_edges, W1, W2, Wd1, bd1, Wd2, bd2):
    i32 = jnp.int32
    rs, rd = edge_index[0, :E_REM], edge_index[1, :E_REM]
    ps, pd = edge_index[0, E_REM:], edge_index[1, E_REM:]
    neg_idx = jax.random.randint(
        jax.random.key(1), (N_MASK,), 0, all_negative_edges.shape[1])

    pad_rem = jnp.full((E_REM_PAD - E_REM,), PAD_NODE, i32)
    rs_p = jnp.concatenate([rs, pad_rem])
    rd_p = jnp.concatenate([rd, pad_rem])
    nit = jnp.concatenate(
        [neg_idx.astype(i32),
         jnp.full((HALF_PAD - N_MASK,), all_negative_edges.shape[1], i32)])
    nst = jnp.concatenate([all_negative_edges[0], jnp.full((8,), PAD_NODE, i32)])
    ndt = jnp.concatenate([all_negative_edges[1], jnp.full((8,), PAD_NODE, i32)])
    x_p = jnp.concatenate([x, jnp.zeros((NP - N_NODES, D), jnp.float32)])
    zflat = jnp.zeros((ROWS_PER_TILE,), jnp.float32)
    z2d = jnp.zeros((128, D), jnp.float32)

    deg_parts, negsrc, negdst = _sc_deg_neg(rd_p, nit, nst, ndt, zflat)

    grid10 = (NP // 1024,)
    row_spec = pl.BlockSpec((1024, 1), lambda i: (i, 0))
    mat_spec = pl.BlockSpec((1024, D), lambda i: (i, 0))
    full128 = pl.BlockSpec((D, D), lambda i: (0, 0))

    d0 = deg_parts[:NP].reshape(NP, 1)
    d1 = deg_parts[NP:].reshape(NP, 1)
    dinv, hn0 = pl.pallas_call(
        _tc_norm_body,
        grid=grid10,
        in_specs=[row_spec, row_spec, mat_spec],
        out_specs=(row_spec, mat_spec),
        out_shape=(jax.ShapeDtypeStruct((NP, 1), jnp.float32),
                   jax.ShapeDtypeStruct((NP, D), jnp.float32)),
    )(d0, d1, x_p)

    def layer(hn, W, relu):
        tmp = _sc_scatter_rows(hn, rs_p, rd_p, z2d)
        return pl.pallas_call(
            functools.partial(_tc_layer_body, relu),
            grid=grid10,
            in_specs=[mat_spec, mat_spec, mat_spec, row_spec, full128],
            out_specs=mat_spec,
            out_shape=jax.ShapeDtypeStruct((NP, D), jnp.float32),
        )(tmp[:NP], tmp[NP:], hn, dinv, W)

    hn1 = layer(hn0, W1, True)
    z = layer(hn1, W2, False)

    pad_half = jnp.full((HALF_PAD - N_MASK,), PAD_NODE, i32)
    dsrc = jnp.concatenate([ps, pad_half, negsrc])
    ddst = jnp.concatenate([pd, pad_half, negdst])
    fs, fd = _sc_dec_gather(z, dsrc, ddst)

    loss = pl.pallas_call(
        _tc_dec_body,
        grid=(DEC_TOTAL // 1024,),
        in_specs=[mat_spec, mat_spec, full128,
                  pl.BlockSpec((1, D), lambda i: (0, 0)),
                  pl.BlockSpec((D, 1), lambda i: (0, 0)),
                  pl.BlockSpec((1, 1), lambda i: (0, 0))],
        out_specs=pl.BlockSpec((1, 1), lambda i: (0, 0)),
        out_shape=jax.ShapeDtypeStruct((1, 1), jnp.float32),
        scratch_shapes=[pltpu.VMEM((1, 2), jnp.float32)],
    )(fs, fd, Wd1, bd1.reshape(1, D), Wd2, bd2.reshape(1, 1))

    return jnp.reshape(loss, ())
